# baseline (speedup 1.0000x reference)
import jax
import jax.numpy as jnp
from jax import lax
from jax.experimental import pallas as pl
from jax.experimental.pallas import tpu as pltpu
from jax.experimental.pallas import tpu_sc as plsc

NC = 2
NS = 16
NW = NC * NS
BSZ = 512
NBUF = 4


def _make_sc_agg(n_pad, n_blocks, with_cnt):
  rpt = n_pad // NS
  assert n_blocks % NBUF == 0 and n_blocks >= NBUF
  mesh = plsc.VectorSubcoreMesh(core_axis_name="c", subcore_axis_name="s")

  out_type = [jax.ShapeDtypeStruct((NC, n_pad, 16), jnp.float32)]
  scratch = (
      [pltpu.VMEM((n_blocks, BSZ), jnp.int32)] * 2
      + [pltpu.VMEM((BSZ, 16), jnp.float32)] * NBUF
      + [pltpu.VMEM((rpt, 16), jnp.float32)]
      + [pltpu.VMEM_SHARED((n_pad, 16), jnp.float32)]
      + [pltpu.SemaphoreType.DMA] * (2 * NBUF)
  )
  if with_cnt:
    out_type.append(jax.ShapeDtypeStruct((NC, n_pad, 16), jnp.float32))
    scratch += [
        pltpu.VMEM((BSZ, 16), jnp.float32),
        pltpu.VMEM_SHARED((n_pad, 16), jnp.float32),
        pltpu.SemaphoreType.DMA,
    ]

  def body(table, src3, dst3, *rest):
    if with_cnt:
      out_f, out_c = rest[0], rest[1]
      rest = rest[2:]
      ones_v, cnt_s, csem = rest[-3:]
    else:
      out_f = rest[0]
      rest = rest[1:]
      out_c = ones_v = cnt_s = csem = None
    src_v, dst_v = rest[0], rest[1]
    rows = rest[2:2 + NBUF]
    stripe_v = rest[2 + NBUF]
    acc_s = rest[3 + NBUF]
    gsem = rest[4 + NBUF:4 + 2 * NBUF]
    ssem = rest[4 + 2 * NBUF:4 + 3 * NBUF]
    c = lax.axis_index("c")
    s = lax.axis_index("s")
    w = s * NC + c

    def zrow(i, _):
      stripe_v[i, :] = jnp.zeros((16,), jnp.float32)
      return _
    lax.fori_loop(0, rpt, zrow, None)
    pltpu.sync_copy(stripe_v, acc_s.at[pl.ds(s * rpt, rpt)])
    if with_cnt:
      pltpu.sync_copy(stripe_v, cnt_s.at[pl.ds(s * rpt, rpt)])

      def orow(i, _):
        ones_v[i, :] = jnp.ones((16,), jnp.float32)
        return _
      lax.fori_loop(0, BSZ, orow, None)

    pltpu.sync_copy(src3.at[w], src_v)
    pltpu.sync_copy(dst3.at[w], dst_v)
    plsc.subcore_barrier()

    def gath(bi, b):
      return pltpu.make_async_copy(table.at[src_v.at[bi]], rows[b], gsem[b])

    def scat(bi, b):
      return pltpu.make_async_copy(rows[b], acc_s.at[dst_v.at[bi]], ssem[b])

    def cscat(bi):
      return pltpu.make_async_copy(ones_v, cnt_s.at[dst_v.at[bi]], csem)

    gath(0, 0).start()
    gath(1, 1).start()

    def quad(qi, _):
      for b in range(NBUF):
        bi = qi * NBUF + b
        nb = (b + 2) % NBUF

        @pl.when(bi >= 2)
        def _():
          scat(bi - 2, nb).wait()

        @pl.when(bi + 2 < n_blocks)
        def _():
          gath(bi + 2, nb).start()

        gath(bi, b).wait()
        scat(bi, b).start(add=True)
        if with_cnt:
          cscat(bi).start(add=True)

          @pl.when(bi >= NBUF)
          def _():
            cscat(bi - NBUF).wait()
      return _
    lax.fori_loop(0, n_blocks // NBUF, quad, None)

    scat(n_blocks - 2, (n_blocks - 2) % NBUF).wait()
    scat(n_blocks - 1, (n_blocks - 1) % NBUF).wait()
    if with_cnt:
      for k in range(NBUF):
        cscat(n_blocks - NBUF + k).wait()
    plsc.subcore_barrier()

    sl = pl.ds(s * rpt, rpt)
    pltpu.sync_copy(acc_s.at[sl], out_f.at[c, sl])
    if with_cnt:
      pltpu.sync_copy(cnt_s.at[sl], out_c.at[c, sl])

  return pl.kernel(
      body, out_type, mesh=mesh, scratch_types=scratch,
      compiler_params=pltpu.CompilerParams(use_tc_tiling_on_sc=False))


def _mm_body(x_ref, w_ref, o_ref):
  o_ref[...] = jnp.dot(x_ref[...], w_ref[...],
                       preferred_element_type=jnp.float32)


def _hidden(accf_ref, accc_ref, xr_ref, b1_ref):
  f = accf_ref[0] + accf_ref[1]
  cnt = accc_ref[0] + accc_ref[1]
  mean = f / jnp.maximum(cnt, 1.0)
  return jnp.maximum(mean + b1_ref[...] + xr_ref[...], 0.0), cnt


def _tc2a_body(accf_ref, accc_ref, xr_ref, b1_ref, wl_ref, hp_ref):
  h, _ = _hidden(accf_ref, accc_ref, xr_ref, b1_ref)
  hp_ref[...] = jnp.dot(h, wl_ref[...], preferred_element_type=jnp.float32)


def _tc2b_body(accf_ref, accc_ref, xr_ref, b1_ref, wr_ref, hr_ref, cnt_ref):
  h, cnt = _hidden(accf_ref, accc_ref, xr_ref, b1_ref)
  hr_ref[...] = jnp.dot(h, wr_ref[...], preferred_element_type=jnp.float32)
  cnt_ref[...] = cnt


def _tc3_body(accf_ref, cnt_ref, hr_ref, b2_ref, out_ref):
  f = accf_ref[0] + accf_ref[1]
  z = f / jnp.maximum(cnt_ref[...], 1.0) + b2_ref[...] + hr_ref[...]
  m = jnp.max(z, axis=1, keepdims=True)
  zs = z - m
  out_ref[...] = zs - jnp.log(jnp.sum(jnp.exp(zs), axis=1, keepdims=True))


def kernel(x, edge_index, W1l, b1, W1r, W2l, b2, W2r):
  n, f_in = x.shape
  e = edge_index.shape[1]
  h = W1l.shape[1]
  assert h == 16 and W2l.shape[1] == 16

  n_pad = 10240
  assert n <= n_pad - 1
  blk = 1024
  grid = n_pad // blk

  per_tile = -(-e // NW)
  n_blocks = -(-per_tile // BSZ)
  n_blocks += -n_blocks % NBUF
  e_pad = NW * n_blocks * BSZ
  src = edge_index[0]
  dst = edge_index[1]
  pad = e_pad - e
  src3 = jnp.concatenate(
      [src, jnp.zeros((pad,), jnp.int32)]).reshape(NW, n_blocks, BSZ)
  dst3 = jnp.concatenate(
      [dst, jnp.full((pad,), n, jnp.int32)]).reshape(NW, n_blocks, BSZ)

  x_pad = jnp.pad(x, ((0, n_pad - n), (0, 0)))
  b1r = b1.reshape(1, h)
  b2r = b2.reshape(1, W2l.shape[1])

  full_spec = pl.BlockSpec((blk, h), lambda i: (i, 0))
  acc_spec = pl.BlockSpec((NC, blk, h), lambda i: (0, i, 0))
  w2_spec = pl.BlockSpec((h, h), lambda i: (0, 0))
  b_spec = pl.BlockSpec((1, h), lambda i: (0, 0))
  x_spec = pl.BlockSpec((blk, f_in), lambda i: (i, 0))
  w1_spec = pl.BlockSpec((f_in, h), lambda i: (0, 0))
  row_shape = jax.ShapeDtypeStruct((n_pad, h), jnp.float32)

  def mm1(w):
    return pl.pallas_call(
        _mm_body, grid=(grid,), in_specs=[x_spec, w1_spec],
        out_specs=full_spec, out_shape=row_shape)(x_pad, w)

  xp1 = mm1(W1l)
  xr1 = mm1(W1r)

  agg1 = _make_sc_agg(n_pad, n_blocks, with_cnt=True)
  acc1, acc_cnt = agg1(xp1, src3, dst3)

  hp2 = pl.pallas_call(
      _tc2a_body, grid=(grid,),
      in_specs=[acc_spec, acc_spec, full_spec, b_spec, w2_spec],
      out_specs=full_spec, out_shape=row_shape,
  )(acc1, acc_cnt, xr1, b1r, W2l)
  hr2, cnt16 = pl.pallas_call(
      _tc2b_body, grid=(grid,),
      in_specs=[acc_spec, acc_spec, full_spec, b_spec, w2_spec],
      out_specs=[full_spec] * 2, out_shape=[row_shape] * 2,
  )(acc1, acc_cnt, xr1, b1r, W2r)

  agg2 = _make_sc_agg(n_pad, n_blocks, with_cnt=False)
  (acc2,) = agg2(hp2, src3, dst3)

  out = pl.pallas_call(
      _tc3_body, grid=(grid,),
      in_specs=[acc_spec, full_spec, full_spec, b_spec],
      out_specs=full_spec, out_shape=row_shape,
  )(acc2, cnt16, hr2, b2r)

  return out[:n]


# speedup vs baseline: 1.7057x; 1.7057x over previous
import jax
import jax.numpy as jnp
from jax import lax
from jax.experimental import pallas as pl
from jax.experimental.pallas import tpu as pltpu
from jax.experimental.pallas import tpu_sc as plsc

NC = 2
NS = 16
NW = NC * NS
BSZ = 1024
NBUF = 3


def _make_sc_agg(n, per_tile, with_cnt):
  n_full, tail = divmod(per_tile, BSZ)
  total = n_full + (1 if tail else 0)
  epi = (total % NBUF) + NBUF
  n_loop = total - epi
  assert n_loop >= 0 and n_loop % NBUF == 0 and total >= 2
  assert tail % 8 == 0
  rpt = -(-n // NS)
  rpt += -rpt % 8
  last_rows = n - (NS - 1) * rpt
  assert 0 < last_rows <= rpt
  mesh = plsc.VectorSubcoreMesh(core_axis_name="c", subcore_axis_name="s")

  out_type = [jax.ShapeDtypeStruct((NC, n, 16), jnp.float32)]
  scratch = (
      [pltpu.VMEM((per_tile,), jnp.int32)] * 2
      + [pltpu.VMEM((BSZ, 16), jnp.float32)] * NBUF
      + [pltpu.VMEM((rpt, 16), jnp.float32)]
      + [pltpu.VMEM_SHARED((n, 16), jnp.float32)]
      + [pltpu.SemaphoreType.DMA] * (2 * NBUF)
  )
  if with_cnt:
    out_type.append(jax.ShapeDtypeStruct((NC, n, 16), jnp.float32))
    scratch += [
        pltpu.VMEM((BSZ, 16), jnp.float32),
        pltpu.VMEM_SHARED((n, 16), jnp.float32),
        pltpu.SemaphoreType.DMA,
    ]

  def body(table, ei, *rest):
    if with_cnt:
      out_f, out_c = rest[0], rest[1]
      rest = rest[2:]
      ones_v, cnt_s, csem = rest[-3:]
    else:
      out_f = rest[0]
      rest = rest[1:]
      out_c = ones_v = cnt_s = csem = None
    src_v, dst_v = rest[0], rest[1]
    rows = rest[2:2 + NBUF]
    stripe_v = rest[2 + NBUF]
    acc_s = rest[3 + NBUF]
    gsem = rest[4 + NBUF:4 + 2 * NBUF]
    ssem = rest[4 + 2 * NBUF:4 + 3 * NBUF]
    c = lax.axis_index("c")
    s = lax.axis_index("s")
    w = s * NC + c

    def zrow(i, _):
      stripe_v[i, :] = jnp.zeros((16,), jnp.float32)
      return _
    lax.fori_loop(0, rpt, zrow, None)

    def zero_acc(acc):
      @pl.when(s < NS - 1)
      def _():
        pltpu.sync_copy(stripe_v, acc.at[pl.ds(s * rpt, rpt)])

      @pl.when(s == NS - 1)
      def _():
        pltpu.sync_copy(stripe_v.at[pl.ds(0, last_rows)],
                        acc.at[pl.ds((NS - 1) * rpt, last_rows)])

    zero_acc(acc_s)
    if with_cnt:
      zero_acc(cnt_s)

      def orow(i, _):
        ones_v[i, :] = jnp.ones((16,), jnp.float32)
        return _
      lax.fori_loop(0, BSZ, orow, None)

    base = w * per_tile
    pltpu.sync_copy(ei.at[0, pl.ds(base, per_tile)], src_v)
    pltpu.sync_copy(ei.at[1, pl.ds(base, per_tile)], dst_v)
    plsc.subcore_barrier()

    def blk_sz(bi):
      return tail if (tail and bi == total - 1) else BSZ

    def rbuf(b, sz):
      return rows[b] if sz == BSZ else rows[b].at[pl.ds(0, sz)]

    def gath(bi, b, sz=BSZ):
      return pltpu.make_async_copy(
          table.at[src_v.at[pl.ds(bi * BSZ, sz)]], rbuf(b, sz), gsem[b])

    def scat(bi, b, sz=BSZ):
      return pltpu.make_async_copy(
          rbuf(b, sz), acc_s.at[dst_v.at[pl.ds(bi * BSZ, sz)]], ssem[b])

    def cscat(bi, sz=BSZ):
      ones = ones_v if sz == BSZ else ones_v.at[pl.ds(0, sz)]
      return pltpu.make_async_copy(
          ones, cnt_s.at[dst_v.at[pl.ds(bi * BSZ, sz)]], csem)

    gath(0, 0).start()
    gath(1, 1).start()

    def maybe(bi, cond, fn):
      if isinstance(bi, int):
        if cond:
          fn()
      else:
        pl.when(cond)(fn)

    def step(bi, b, nb, sz, ahead_sz):
      maybe(bi, bi >= NBUF - 2, lambda: scat(bi + 2 - NBUF, nb).wait())
      if ahead_sz:
        gath(bi + 2, nb, ahead_sz).start()
      gath(bi, b, sz).wait()
      scat(bi, b, sz).start(add=True)
      if with_cnt:
        cscat(bi, sz).start(add=True)
        maybe(bi, bi >= NBUF, lambda: cscat(bi - NBUF).wait())

    def quad(qi, _):
      for b in range(NBUF):
        bi = qi * NBUF + b
        step(bi, b, (b + 2) % NBUF, BSZ, ahead_sz=BSZ)
      return _
    lax.fori_loop(0, n_loop // NBUF, quad, None)

    for bi in range(n_loop, total):
      b = bi % NBUF
      ahead = blk_sz(bi + 2) if bi + 2 < total else None
      step(bi, b, (b + 2) % NBUF, blk_sz(bi), ahead_sz=ahead)

    for bi in range(total - (NBUF - 2), total):
      scat(bi, bi % NBUF, blk_sz(bi)).wait()
    if with_cnt:
      for bi in range(max(0, total - NBUF), total):
        cscat(bi, blk_sz(bi)).wait()
    plsc.subcore_barrier()

    def writeback(acc, out):
      @pl.when(s < NS - 1)
      def _():
        sl = pl.ds(s * rpt, rpt)
        pltpu.sync_copy(acc.at[sl], out.at[c, sl])

      @pl.when(s == NS - 1)
      def _():
        sl = pl.ds((NS - 1) * rpt, last_rows)
        pltpu.sync_copy(acc.at[sl], out.at[c, sl])

    writeback(acc_s, out_f)
    if with_cnt:
      writeback(cnt_s, out_c)

  return pl.kernel(
      body, out_type, mesh=mesh, scratch_types=scratch,
      compiler_params=pltpu.CompilerParams(use_tc_tiling_on_sc=False))


def _tc1_body(x_ref, wl_ref, wr_ref, pl_ref, pr_ref):
  x = x_ref[...]
  pl_ref[...] = jnp.dot(x, wl_ref[...], preferred_element_type=jnp.float32)
  pr_ref[...] = jnp.dot(x, wr_ref[...], preferred_element_type=jnp.float32)


def _tc2_body(accf_ref, accc_ref, xr_ref, b1_ref, wl_ref, wr_ref,
              hp_ref, hr_ref, cnt_ref):
  f = accf_ref[0] + accf_ref[1]
  cnt = accc_ref[0] + accc_ref[1]
  mean = f / jnp.maximum(cnt, 1.0)
  h = jnp.maximum(mean + b1_ref[...] + xr_ref[...], 0.0)
  hp_ref[...] = jnp.dot(h, wl_ref[...], preferred_element_type=jnp.float32)
  hr_ref[...] = jnp.dot(h, wr_ref[...], preferred_element_type=jnp.float32)
  cnt_ref[...] = cnt


def _tc3_body(accf_ref, cnt_ref, hr_ref, b2_ref, out_ref):
  f = accf_ref[0] + accf_ref[1]
  z = f / jnp.maximum(cnt_ref[...], 1.0) + b2_ref[...] + hr_ref[...]
  m = jnp.max(z, axis=1, keepdims=True)
  zs = z - m
  out_ref[...] = zs - jnp.log(jnp.sum(jnp.exp(zs), axis=1, keepdims=True))


def kernel(x, edge_index, W1l, b1, W1r, W2l, b2, W2r):
  n, f_in = x.shape
  e = edge_index.shape[1]
  h = W1l.shape[1]
  assert h == 16 and W2l.shape[1] == 16

  per_tile, rem = divmod(e, NW)
  assert rem == 0 and per_tile % 8 == 0

  blk = 1000
  assert n % blk == 0 and blk % 8 == 0
  grid = n // blk

  b1r = b1.reshape(1, h)
  b2r = b2.reshape(1, W2l.shape[1])
  ei = edge_index.astype(jnp.int32)

  full_spec = pl.BlockSpec((blk, h), lambda i: (i, 0))
  acc_spec = pl.BlockSpec((NC, blk, h), lambda i: (0, i, 0))
  w2_spec = pl.BlockSpec((h, h), lambda i: (0, 0))
  b_spec = pl.BlockSpec((1, h), lambda i: (0, 0))
  x_spec = pl.BlockSpec((blk, f_in), lambda i: (i, 0))
  w1_spec = pl.BlockSpec((f_in, h), lambda i: (0, 0))
  row_shape = jax.ShapeDtypeStruct((n, h), jnp.float32)

  xp1, xr1 = pl.pallas_call(
      _tc1_body, grid=(grid,), in_specs=[x_spec, w1_spec, w1_spec],
      out_specs=[full_spec] * 2, out_shape=[row_shape] * 2)(x, W1l, W1r)

  agg1 = _make_sc_agg(n, per_tile, with_cnt=True)
  acc1, acc_cnt = agg1(xp1, ei)

  hp2, hr2, cnt16 = pl.pallas_call(
      _tc2_body, grid=(grid,),
      in_specs=[acc_spec, acc_spec, full_spec, b_spec, w2_spec, w2_spec],
      out_specs=[full_spec] * 3, out_shape=[row_shape] * 3,
  )(acc1, acc_cnt, xr1, b1r, W2l, W2r)

  agg2 = _make_sc_agg(n, per_tile, with_cnt=False)
  (acc2,) = agg2(hp2, ei)

  out = pl.pallas_call(
      _tc3_body, grid=(grid,),
      in_specs=[acc_spec, full_spec, full_spec, b_spec],
      out_specs=full_spec, out_shape=row_shape,
  )(acc2, cnt16, hr2, b2r)

  return out


# speedup vs baseline: 2.3300x; 1.3660x over previous
import jax
import jax.numpy as jnp
from jax import lax
from jax.experimental import pallas as pl
from jax.experimental.pallas import tpu as pltpu
from jax.experimental.pallas import tpu_sc as plsc

NC = 2
NS = 16
NW = NC * NS
BSZ = 1024
NBUF = 3


def _make_sc_agg(n, per_tile, e, with_cnt):
  n_full, tail = divmod(per_tile, BSZ)
  total = n_full + (1 if tail else 0)
  epi = (total % NBUF) + NBUF
  n_loop = total - epi
  assert n_loop >= 0 and n_loop % NBUF == 0 and total >= 2
  assert tail % 8 == 0
  rpt = -(-n // NS)
  rpt += -rpt % 8
  last_rows = n - (NS - 1) * rpt
  assert 0 < last_rows <= rpt
  mesh = plsc.VectorSubcoreMesh(core_axis_name="c", subcore_axis_name="s")

  out_type = [jax.ShapeDtypeStruct((NC, n, 16), jnp.float32)]
  scratch = (
      [pltpu.VMEM((per_tile,), jnp.int32)] * 2
      + [pltpu.VMEM((BSZ, 16), jnp.float32)] * NBUF
      + [pltpu.VMEM((rpt, 16), jnp.float32)]
      + [pltpu.VMEM_SHARED((n, 16), jnp.float32)]
      + [pltpu.SemaphoreType.DMA] * (2 * NBUF)
  )
  if with_cnt:
    out_type.append(jax.ShapeDtypeStruct((NC, n, 16), jnp.float32))
    scratch += [
        pltpu.VMEM((BSZ, 16), jnp.float32),
        pltpu.VMEM_SHARED((n, 16), jnp.float32),
        pltpu.SemaphoreType.DMA,
    ]

  def body(table, ei, *rest):
    if with_cnt:
      out_f, out_c = rest[0], rest[1]
      rest = rest[2:]
      ones_v, cnt_s, csem = rest[-3:]
    else:
      out_f = rest[0]
      rest = rest[1:]
      out_c = ones_v = cnt_s = csem = None
    src_v, dst_v = rest[0], rest[1]
    rows = rest[2:2 + NBUF]
    stripe_v = rest[2 + NBUF]
    acc_s = rest[3 + NBUF]
    gsem = rest[4 + NBUF:4 + 2 * NBUF]
    ssem = rest[4 + 2 * NBUF:4 + 3 * NBUF]
    c = lax.axis_index("c")
    s = lax.axis_index("s")
    w = s * NC + c

    def zrow(i, _):
      stripe_v[i, :] = jnp.zeros((16,), jnp.float32)
      return _
    lax.fori_loop(0, rpt, zrow, None)

    def zero_acc(acc):
      @pl.when(s < NS - 1)
      def _():
        pltpu.sync_copy(stripe_v, acc.at[pl.ds(s * rpt, rpt)])

      @pl.when(s == NS - 1)
      def _():
        pltpu.sync_copy(stripe_v.at[pl.ds(0, last_rows)],
                        acc.at[pl.ds((NS - 1) * rpt, last_rows)])

    zero_acc(acc_s)
    if with_cnt:
      zero_acc(cnt_s)

      def orow(i, _):
        ones_v[i, :] = jnp.ones((16,), jnp.float32)
        return _
      lax.fori_loop(0, BSZ, orow, None)

    base = w * per_tile
    pltpu.sync_copy(ei.at[pl.ds(base, per_tile)], src_v)
    pltpu.sync_copy(ei.at[pl.ds(e + base, per_tile)], dst_v)
    plsc.subcore_barrier()

    def blk_sz(bi):
      return tail if (tail and bi == total - 1) else BSZ

    def rbuf(b, sz):
      return rows[b] if sz == BSZ else rows[b].at[pl.ds(0, sz)]

    def gath(bi, b, sz=BSZ):
      return pltpu.make_async_copy(
          table.at[src_v.at[pl.ds(bi * BSZ, sz)]], rbuf(b, sz), gsem[b])

    def scat(bi, b, sz=BSZ):
      return pltpu.make_async_copy(
          rbuf(b, sz), acc_s.at[dst_v.at[pl.ds(bi * BSZ, sz)]], ssem[b])

    def cscat(bi, sz=BSZ):
      ones = ones_v if sz == BSZ else ones_v.at[pl.ds(0, sz)]
      return pltpu.make_async_copy(
          ones, cnt_s.at[dst_v.at[pl.ds(bi * BSZ, sz)]], csem)

    gath(0, 0).start()
    gath(1, 1).start()

    def maybe(bi, cond, fn):
      if isinstance(bi, int):
        if cond:
          fn()
      else:
        pl.when(cond)(fn)

    def step(bi, b, nb, sz, ahead_sz):
      maybe(bi, bi >= NBUF - 2, lambda: scat(bi + 2 - NBUF, nb).wait())
      if ahead_sz:
        gath(bi + 2, nb, ahead_sz).start()
      gath(bi, b, sz).wait()
      scat(bi, b, sz).start(add=True)
      if with_cnt:
        cscat(bi, sz).start(add=True)
        maybe(bi, bi >= NBUF, lambda: cscat(bi - NBUF).wait())

    def quad(qi, _):
      for b in range(NBUF):
        bi = qi * NBUF + b
        step(bi, b, (b + 2) % NBUF, BSZ, ahead_sz=BSZ)
      return _
    lax.fori_loop(0, n_loop // NBUF, quad, None)

    for bi in range(n_loop, total):
      b = bi % NBUF
      ahead = blk_sz(bi + 2) if bi + 2 < total else None
      step(bi, b, (b + 2) % NBUF, blk_sz(bi), ahead_sz=ahead)

    for bi in range(total - (NBUF - 2), total):
      scat(bi, bi % NBUF, blk_sz(bi)).wait()
    if with_cnt:
      for bi in range(max(0, total - NBUF), total):
        cscat(bi, blk_sz(bi)).wait()
    plsc.subcore_barrier()

    def writeback(acc, out):
      @pl.when(s < NS - 1)
      def _():
        sl = pl.ds(s * rpt, rpt)
        pltpu.sync_copy(acc.at[sl], out.at[c, sl])

      @pl.when(s == NS - 1)
      def _():
        sl = pl.ds((NS - 1) * rpt, last_rows)
        pltpu.sync_copy(acc.at[sl], out.at[c, sl])

    writeback(acc_s, out_f)
    if with_cnt:
      writeback(cnt_s, out_c)

  return pl.kernel(
      body, out_type, mesh=mesh, scratch_types=scratch,
      compiler_params=pltpu.CompilerParams(use_tc_tiling_on_sc=False))


def _tc1_body(xg_ref, bl_ref, br_ref, pl_ref, pr_ref):
  xg = xg_ref[...]
  pl_ref[...] = jnp.dot(xg, bl_ref[...], preferred_element_type=jnp.float32)
  pr_ref[...] = jnp.dot(xg, br_ref[...], preferred_element_type=jnp.float32)


def _tc2_body(accf_ref, accc_ref, xr_ref, b1_ref, wl_ref, wr_ref,
              hp_ref, hr_ref):
  f = accf_ref[0] + accf_ref[1]
  cnt = accc_ref[0] + accc_ref[1]
  mean = f / jnp.maximum(cnt, 1.0)
  h = jnp.maximum(mean + b1_ref[...] + xr_ref[...], 0.0)
  hp_ref[...] = jnp.dot(h, wl_ref[...], preferred_element_type=jnp.float32)
  hr_ref[...] = jnp.dot(h, wr_ref[...], preferred_element_type=jnp.float32)


def _tc3_body(accf_ref, accc_ref, hr_ref, b2_ref, onesbd_ref, out_ref):
  f = accf_ref[0] + accf_ref[1]
  cnt = accc_ref[0] + accc_ref[1]
  z = f / jnp.maximum(cnt, 1.0) + b2_ref[...] + hr_ref[...]
  off = jax.lax.broadcasted_iota(jnp.int32, z.shape, 1) % 16
  m = z
  for sft in (1, 2, 4, 8):
    right = pltpu.roll(m, 128 - sft, 1)
    left = pltpu.roll(m, sft, 1)
    m = jnp.where(off + sft < 16, jnp.maximum(m, right), m)
    m = jnp.where(off >= sft, jnp.maximum(m, left), m)
  zs = z - m
  e = jnp.exp(zs)
  s_all = jnp.dot(e, onesbd_ref[...], preferred_element_type=jnp.float32)
  out_ref[...] = zs - jnp.log(s_all)


def _block_diag8(w):
  return jnp.einsum("jJ,kf->jkJf", jnp.eye(8, dtype=w.dtype), w).reshape(
      8 * w.shape[0], 8 * w.shape[1])


def kernel(x, edge_index, W1l, b1, W1r, W2l, b2, W2r):
  n, f_in = x.shape
  e = edge_index.shape[1]
  h = W1l.shape[1]
  assert h == 16 and W2l.shape[1] == 16 and n % 8 == 0

  per_tile, rem = divmod(e, NW)
  assert rem == 0 and per_tile % 8 == 0

  npk = n // 8
  blk = npk
  grid = 1

  ei = edge_index.astype(jnp.int32).reshape(2 * e)
  xg = x.reshape(npk, 8 * f_in)
  B1l = jnp.einsum("jJ,kf->jkJf", jnp.eye(8, dtype=x.dtype),
                   W1l).reshape(8 * f_in, 8 * h)
  B1r = jnp.einsum("jJ,kf->jkJf", jnp.eye(8, dtype=x.dtype),
                   W1r).reshape(8 * f_in, 8 * h)
  Wd2l = _block_diag8(W2l)
  Wd2r = _block_diag8(W2r)
  b1t = jnp.tile(b1, 8).reshape(1, 8 * h)
  b2t = jnp.tile(b2, 8).reshape(1, 8 * h)

  pk_spec = pl.BlockSpec((blk, 128), lambda i: (i, 0))
  acc_spec = pl.BlockSpec((NC, blk, 128), lambda i: (0, i, 0))
  wd_spec = pl.BlockSpec((128, 128), lambda i: (0, 0))
  b_spec = pl.BlockSpec((1, 128), lambda i: (0, 0))
  xg_spec = pl.BlockSpec((blk, 8 * f_in), lambda i: (i, 0))
  w1_spec = pl.BlockSpec((8 * f_in, 128), lambda i: (0, 0))
  pk_shape = jax.ShapeDtypeStruct((npk, 128), jnp.float32)

  xp1, xr1 = pl.pallas_call(
      _tc1_body, grid=(grid,), in_specs=[xg_spec, w1_spec, w1_spec],
      out_specs=[pk_spec] * 2, out_shape=[pk_shape] * 2)(xg, B1l, B1r)

  agg1 = _make_sc_agg(n, per_tile, e, with_cnt=True)
  acc1, acc_cnt = agg1(xp1.reshape(n, h), ei)
  acc1 = acc1.reshape(NC, npk, 128)
  acc_cnt = acc_cnt.reshape(NC, npk, 128)

  hp2, hr2 = pl.pallas_call(
      _tc2_body, grid=(grid,),
      in_specs=[acc_spec, acc_spec, pk_spec, b_spec, wd_spec, wd_spec],
      out_specs=[pk_spec] * 2, out_shape=[pk_shape] * 2,
  )(acc1, acc_cnt, xr1, b1t, Wd2l, Wd2r)

  agg2 = _make_sc_agg(n, per_tile, e, with_cnt=False)
  (acc2,) = agg2(hp2.reshape(n, h), ei)
  acc2 = acc2.reshape(NC, npk, 128)

  onesbd = _block_diag8(jnp.ones((h, h), jnp.float32))
  out_pk = pl.pallas_call(
      _tc3_body, grid=(grid,),
      in_specs=[acc_spec, acc_spec, pk_spec, b_spec, wd_spec],
      out_specs=pk_spec, out_shape=pk_shape,
  )(acc2, acc_cnt, hr2, b2t, onesbd)

  return out_pk.reshape(n, h)


# speedup vs baseline: 2.5207x; 1.0819x over previous
import jax
import jax.numpy as jnp
from jax import lax
from jax.experimental import pallas as pl
from jax.experimental.pallas import tpu as pltpu
from jax.experimental.pallas import tpu_sc as plsc

NC = 2
NS = 16
NW = NC * NS
BSZ = 1024
NBUF = 3


def _make_sc_agg(n, per_tile, e, with_cnt):
  n_full, tail = divmod(per_tile, BSZ)
  total = n_full + (1 if tail else 0)
  epi = (total % NBUF) + NBUF
  n_loop = total - epi
  assert n_loop >= 0 and n_loop % NBUF == 0 and total >= 2
  assert tail % 8 == 0
  rpt = -(-n // NS)
  rpt += -rpt % 8
  last_rows = n - (NS - 1) * rpt
  assert 0 < last_rows <= rpt
  mesh = plsc.VectorSubcoreMesh(core_axis_name="c", subcore_axis_name="s")

  out_type = [jax.ShapeDtypeStruct((NC, n, 16), jnp.float32)]
  scratch = (
      [pltpu.VMEM((per_tile,), jnp.int32)] * 2
      + [pltpu.VMEM((BSZ, 16), jnp.float32)] * NBUF
      + [pltpu.VMEM((rpt, 16), jnp.float32)]
      + [pltpu.VMEM_SHARED((n, 16), jnp.float32)]
      + [pltpu.SemaphoreType.DMA] * (2 * NBUF)
  )
  if with_cnt:
    out_type.append(jax.ShapeDtypeStruct((NC, n, 16), jnp.float32))
    scratch += [
        pltpu.VMEM((BSZ, 16), jnp.float32),
        pltpu.VMEM_SHARED((n, 16), jnp.float32),
        pltpu.SemaphoreType.DMA,
    ]

  def body(table, ei, *rest):
    if with_cnt:
      out_f, out_c = rest[0], rest[1]
      rest = rest[2:]
      ones_v, cnt_s, csem = rest[-3:]
    else:
      out_f = rest[0]
      rest = rest[1:]
      out_c = ones_v = cnt_s = csem = None
    src_v, dst_v = rest[0], rest[1]
    rows = rest[2:2 + NBUF]
    stripe_v = rest[2 + NBUF]
    acc_s = rest[3 + NBUF]
    gsem = rest[4 + NBUF:4 + 2 * NBUF]
    ssem = rest[4 + 2 * NBUF:4 + 3 * NBUF]
    c = lax.axis_index("c")
    s = lax.axis_index("s")
    w = s * NC + c

    assert rpt % 8 == 0
    def zrow(i, _):
      for j in range(8):
        stripe_v[i * 8 + j, :] = jnp.zeros((16,), jnp.float32)
      return _
    lax.fori_loop(0, rpt // 8, zrow, None)

    def zero_acc(acc):
      @pl.when(s < NS - 1)
      def _():
        pltpu.sync_copy(stripe_v, acc.at[pl.ds(s * rpt, rpt)])

      @pl.when(s == NS - 1)
      def _():
        pltpu.sync_copy(stripe_v.at[pl.ds(0, last_rows)],
                        acc.at[pl.ds((NS - 1) * rpt, last_rows)])

    zero_acc(acc_s)
    if with_cnt:
      zero_acc(cnt_s)

      def orow(i, _):
        for j in range(8):
          ones_v[i * 8 + j, :] = jnp.ones((16,), jnp.float32)
        return _
      lax.fori_loop(0, BSZ // 8, orow, None)

    base = w * per_tile
    pltpu.sync_copy(ei.at[pl.ds(base, per_tile)], src_v)
    pltpu.sync_copy(ei.at[pl.ds(e + base, per_tile)], dst_v)
    plsc.subcore_barrier()

    def blk_sz(bi):
      return tail if (tail and bi == total - 1) else BSZ

    def rbuf(b, sz):
      return rows[b] if sz == BSZ else rows[b].at[pl.ds(0, sz)]

    def gath(bi, b, sz=BSZ):
      return pltpu.make_async_copy(
          table.at[src_v.at[pl.ds(bi * BSZ, sz)]], rbuf(b, sz), gsem[b])

    def scat(bi, b, sz=BSZ):
      return pltpu.make_async_copy(
          rbuf(b, sz), acc_s.at[dst_v.at[pl.ds(bi * BSZ, sz)]], ssem[b])

    def cscat(bi, sz=BSZ):
      ones = ones_v if sz == BSZ else ones_v.at[pl.ds(0, sz)]
      return pltpu.make_async_copy(
          ones, cnt_s.at[dst_v.at[pl.ds(bi * BSZ, sz)]], csem)

    gath(0, 0).start()
    gath(1, 1).start()

    def maybe(bi, cond, fn):
      if isinstance(bi, int):
        if cond:
          fn()
      else:
        pl.when(cond)(fn)

    def step(bi, b, nb, sz, ahead_sz):
      maybe(bi, bi >= NBUF - 2, lambda: scat(bi + 2 - NBUF, nb).wait())
      if ahead_sz:
        gath(bi + 2, nb, ahead_sz).start()
      gath(bi, b, sz).wait()
      scat(bi, b, sz).start(add=True)
      if with_cnt:
        cscat(bi, sz).start(add=True)
        maybe(bi, bi >= NBUF, lambda: cscat(bi - NBUF).wait())

    def quad(qi, _):
      for b in range(NBUF):
        bi = qi * NBUF + b
        step(bi, b, (b + 2) % NBUF, BSZ, ahead_sz=BSZ)
      return _
    lax.fori_loop(0, n_loop // NBUF, quad, None)

    for bi in range(n_loop, total):
      b = bi % NBUF
      ahead = blk_sz(bi + 2) if bi + 2 < total else None
      step(bi, b, (b + 2) % NBUF, blk_sz(bi), ahead_sz=ahead)

    for bi in range(total - (NBUF - 2), total):
      scat(bi, bi % NBUF, blk_sz(bi)).wait()
    if with_cnt:
      for bi in range(max(0, total - NBUF), total):
        cscat(bi, blk_sz(bi)).wait()
    plsc.subcore_barrier()

    def writeback(acc, out):
      @pl.when(s < NS - 1)
      def _():
        sl = pl.ds(s * rpt, rpt)
        pltpu.sync_copy(acc.at[sl], out.at[c, sl])

      @pl.when(s == NS - 1)
      def _():
        sl = pl.ds((NS - 1) * rpt, last_rows)
        pltpu.sync_copy(acc.at[sl], out.at[c, sl])

    writeback(acc_s, out_f)
    if with_cnt:
      writeback(cnt_s, out_c)

  return pl.kernel(
      body, out_type, mesh=mesh, scratch_types=scratch,
      compiler_params=pltpu.CompilerParams(use_tc_tiling_on_sc=False))


def _tc1_body(xg_ref, bl_ref, br_ref, pl_ref, pr_ref):
  xg = xg_ref[...]
  pl_ref[...] = jnp.dot(xg, bl_ref[...], preferred_element_type=jnp.float32)
  pr_ref[...] = jnp.dot(xg, br_ref[...], preferred_element_type=jnp.float32)


def _tc2_body(accf_ref, accc_ref, xr_ref, b1_ref, wl_ref, wr_ref,
              hp_ref, hr_ref):
  f = accf_ref[0] + accf_ref[1]
  cnt = accc_ref[0] + accc_ref[1]
  mean = f / jnp.maximum(cnt, 1.0)
  h = jnp.maximum(mean + b1_ref[...] + xr_ref[...], 0.0)
  hp_ref[...] = jnp.dot(h, wl_ref[...], preferred_element_type=jnp.float32)
  hr_ref[...] = jnp.dot(h, wr_ref[...], preferred_element_type=jnp.float32)


def _tc3_body(accf_ref, accc_ref, hr_ref, b2_ref, onesbd_ref, out_ref):
  f = accf_ref[0] + accf_ref[1]
  cnt = accc_ref[0] + accc_ref[1]
  z = f / jnp.maximum(cnt, 1.0) + b2_ref[...] + hr_ref[...]
  off = jax.lax.broadcasted_iota(jnp.int32, z.shape, 1) % 16
  m = z
  for sft in (1, 2, 4, 8):
    right = pltpu.roll(m, 128 - sft, 1)
    left = pltpu.roll(m, sft, 1)
    m = jnp.where(off + sft < 16, jnp.maximum(m, right), m)
    m = jnp.where(off >= sft, jnp.maximum(m, left), m)
  zs = z - m
  e = jnp.exp(zs)
  s_all = jnp.dot(e, onesbd_ref[...], preferred_element_type=jnp.float32)
  out_ref[...] = zs - jnp.log(s_all)


def _block_diag8(w):
  return jnp.einsum("jJ,kf->jkJf", jnp.eye(8, dtype=w.dtype), w).reshape(
      8 * w.shape[0], 8 * w.shape[1])


def kernel(x, edge_index, W1l, b1, W1r, W2l, b2, W2r):
  n, f_in = x.shape
  e = edge_index.shape[1]
  h = W1l.shape[1]
  assert h == 16 and W2l.shape[1] == 16 and n % 8 == 0

  per_tile, rem = divmod(e, NW)
  assert rem == 0 and per_tile % 8 == 0

  npk = n // 8
  blk = npk
  grid = 1

  ei = edge_index.astype(jnp.int32).reshape(2 * e)
  xg = x.reshape(npk, 8 * f_in)
  B1l = jnp.einsum("jJ,kf->jkJf", jnp.eye(8, dtype=x.dtype),
                   W1l).reshape(8 * f_in, 8 * h)
  B1r = jnp.einsum("jJ,kf->jkJf", jnp.eye(8, dtype=x.dtype),
                   W1r).reshape(8 * f_in, 8 * h)
  Wd2l = _block_diag8(W2l)
  Wd2r = _block_diag8(W2r)
  b1t = jnp.tile(b1, 8).reshape(1, 8 * h)
  b2t = jnp.tile(b2, 8).reshape(1, 8 * h)

  pk_spec = pl.BlockSpec((blk, 128), lambda i: (i, 0))
  acc_spec = pl.BlockSpec((NC, blk, 128), lambda i: (0, i, 0))
  wd_spec = pl.BlockSpec((128, 128), lambda i: (0, 0))
  b_spec = pl.BlockSpec((1, 128), lambda i: (0, 0))
  xg_spec = pl.BlockSpec((blk, 8 * f_in), lambda i: (i, 0))
  w1_spec = pl.BlockSpec((8 * f_in, 128), lambda i: (0, 0))
  pk_shape = jax.ShapeDtypeStruct((npk, 128), jnp.float32)

  xp1, xr1 = pl.pallas_call(
      _tc1_body, grid=(grid,), in_specs=[xg_spec, w1_spec, w1_spec],
      out_specs=[pk_spec] * 2, out_shape=[pk_shape] * 2)(xg, B1l, B1r)

  agg1 = _make_sc_agg(n, per_tile, e, with_cnt=True)
  acc1, acc_cnt = agg1(xp1.reshape(n, h), ei)
  acc1 = acc1.reshape(NC, npk, 128)
  acc_cnt = acc_cnt.reshape(NC, npk, 128)

  hp2, hr2 = pl.pallas_call(
      _tc2_body, grid=(grid,),
      in_specs=[acc_spec, acc_spec, pk_spec, b_spec, wd_spec, wd_spec],
      out_specs=[pk_spec] * 2, out_shape=[pk_shape] * 2,
  )(acc1, acc_cnt, xr1, b1t, Wd2l, Wd2r)

  agg2 = _make_sc_agg(n, per_tile, e, with_cnt=False)
  (acc2,) = agg2(hp2.reshape(n, h), ei)
  acc2 = acc2.reshape(NC, npk, 128)

  onesbd = _block_diag8(jnp.ones((h, h), jnp.float32))
  out_pk = pl.pallas_call(
      _tc3_body, grid=(grid,),
      in_specs=[acc_spec, acc_spec, pk_spec, b_spec, wd_spec],
      out_specs=pk_spec, out_shape=pk_shape,
  )(acc2, acc_cnt, hr2, b2t, onesbd)

  return out_pk.reshape(n, h)


# speedup vs baseline: 2.7464x; 1.0895x over previous
import jax
import jax.numpy as jnp
from jax import lax
from jax.experimental import pallas as pl
from jax.experimental.pallas import tpu as pltpu
from jax.experimental.pallas import tpu_sc as plsc

NC = 2
NS = 16
NW = NC * NS
BSZ = 1024
NBUF = 3


def _make_sc_agg(n, per_tile, e, with_cnt):
  n_full, tail = divmod(per_tile, BSZ)
  total = n_full + (1 if tail else 0)
  epi = (total % NBUF) + NBUF
  n_loop = total - epi
  assert n_loop >= 0 and n_loop % NBUF == 0 and total >= 2
  assert tail % 8 == 0
  rpt = -(-n // NS)
  rpt += -rpt % 8
  last_rows = n - (NS - 1) * rpt
  assert 0 < last_rows <= rpt
  mesh = plsc.VectorSubcoreMesh(core_axis_name="c", subcore_axis_name="s")

  out_type = [jax.ShapeDtypeStruct((NC, n, 16), jnp.float32)]
  scratch = (
      [pltpu.VMEM((per_tile,), jnp.int32)] * 2
      + [pltpu.VMEM((BSZ, 16), jnp.float32)] * NBUF
      + [pltpu.VMEM((rpt, 16), jnp.float32)]
      + [pltpu.VMEM_SHARED((n, 16), jnp.float32)]
      + [pltpu.SemaphoreType.DMA] * (2 * NBUF)
  )
  if with_cnt:
    out_type.append(jax.ShapeDtypeStruct((NC, n, 16), jnp.float32))
    scratch += [
        pltpu.VMEM((BSZ, 16), jnp.float32),
        pltpu.VMEM_SHARED((n, 16), jnp.float32),
        pltpu.SemaphoreType.DMA,
    ]

  def body(table, ei, *rest):
    if with_cnt:
      out_f, out_c = rest[0], rest[1]
      rest = rest[2:]
      ones_v, cnt_s, csem = rest[-3:]
    else:
      out_f = rest[0]
      rest = rest[1:]
      out_c = ones_v = cnt_s = csem = None
    src_v, dst_v = rest[0], rest[1]
    rows = rest[2:2 + NBUF]
    stripe_v = rest[2 + NBUF]
    acc_s = rest[3 + NBUF]
    gsem = rest[4 + NBUF:4 + 2 * NBUF]
    ssem = rest[4 + 2 * NBUF:4 + 3 * NBUF]
    c = lax.axis_index("c")
    s = lax.axis_index("s")
    w = s * NC + c

    assert rpt % 8 == 0
    def zrow(i, _):
      for j in range(8):
        stripe_v[i * 8 + j, :] = jnp.zeros((16,), jnp.float32)
      return _
    lax.fori_loop(0, rpt // 8, zrow, None)

    def zero_acc(acc):
      @pl.when(s < NS - 1)
      def _():
        pltpu.sync_copy(stripe_v, acc.at[pl.ds(s * rpt, rpt)])

      @pl.when(s == NS - 1)
      def _():
        pltpu.sync_copy(stripe_v.at[pl.ds(0, last_rows)],
                        acc.at[pl.ds((NS - 1) * rpt, last_rows)])

    zero_acc(acc_s)
    if with_cnt:
      zero_acc(cnt_s)

      def orow(i, _):
        for j in range(8):
          ones_v[i * 8 + j, :] = jnp.ones((16,), jnp.float32)
        return _
      lax.fori_loop(0, BSZ // 8, orow, None)

    base = w * per_tile
    pltpu.sync_copy(ei.at[pl.ds(base, per_tile)], src_v)
    pltpu.sync_copy(ei.at[pl.ds(e + base, per_tile)], dst_v)
    plsc.subcore_barrier()

    def blk_sz(bi):
      return tail if (tail and bi == total - 1) else BSZ

    def rbuf(b, sz):
      return rows[b] if sz == BSZ else rows[b].at[pl.ds(0, sz)]

    def gath(bi, b, sz=BSZ):
      return pltpu.make_async_copy(
          table.at[src_v.at[pl.ds(bi * BSZ, sz)]], rbuf(b, sz), gsem[b])

    def scat(bi, b, sz=BSZ):
      return pltpu.make_async_copy(
          rbuf(b, sz), acc_s.at[dst_v.at[pl.ds(bi * BSZ, sz)]], ssem[b])

    def cscat(bi, sz=BSZ):
      ones = ones_v if sz == BSZ else ones_v.at[pl.ds(0, sz)]
      return pltpu.make_async_copy(
          ones, cnt_s.at[dst_v.at[pl.ds(bi * BSZ, sz)]], csem)

    gath(0, 0).start()
    gath(1, 1).start()

    def maybe(bi, cond, fn):
      if isinstance(bi, int):
        if cond:
          fn()
      else:
        pl.when(cond)(fn)

    def step(bi, b, nb, sz, ahead_sz):
      maybe(bi, bi >= NBUF - 2, lambda: scat(bi + 2 - NBUF, nb).wait())
      if ahead_sz:
        gath(bi + 2, nb, ahead_sz).start()
      gath(bi, b, sz).wait()
      scat(bi, b, sz).start(add=True)
      if with_cnt:
        cscat(bi, sz).start(add=True)
        maybe(bi, bi >= NBUF, lambda: cscat(bi - NBUF).wait())

    def quad(qi, _):
      for b in range(NBUF):
        bi = qi * NBUF + b
        step(bi, b, (b + 2) % NBUF, BSZ, ahead_sz=BSZ)
      return _
    lax.fori_loop(0, n_loop // NBUF, quad, None)

    for bi in range(n_loop, total):
      b = bi % NBUF
      ahead = blk_sz(bi + 2) if bi + 2 < total else None
      step(bi, b, (b + 2) % NBUF, blk_sz(bi), ahead_sz=ahead)

    for bi in range(total - (NBUF - 2), total):
      scat(bi, bi % NBUF, blk_sz(bi)).wait()
    if with_cnt:
      for bi in range(max(0, total - NBUF), total):
        cscat(bi, blk_sz(bi)).wait()
    plsc.subcore_barrier()

    def writeback(acc, out):
      @pl.when(s < NS - 1)
      def _():
        sl = pl.ds(s * rpt, rpt)
        pltpu.sync_copy(acc.at[sl], out.at[c, sl])

      @pl.when(s == NS - 1)
      def _():
        sl = pl.ds((NS - 1) * rpt, last_rows)
        pltpu.sync_copy(acc.at[sl], out.at[c, sl])

    writeback(acc_s, out_f)
    if with_cnt:
      writeback(cnt_s, out_c)

  return pl.kernel(
      body, out_type, mesh=mesh, scratch_types=scratch,
      compiler_params=pltpu.CompilerParams(use_tc_tiling_on_sc=False))


def _tc1_body(x3_ref, wl_ref, wr_ref, pl_ref, pr_ref):
  wl = wl_ref[...]
  wr = wr_ref[...]
  for j in range(8):
    xj = x3_ref[:, j, :]
    pl_ref[:, j * 16:(j + 1) * 16] = jnp.dot(
        xj, wl, preferred_element_type=jnp.float32)
    pr_ref[:, j * 16:(j + 1) * 16] = jnp.dot(
        xj, wr, preferred_element_type=jnp.float32)


def _tc2_body(accf_ref, accc_ref, xr_ref, b1_ref, wl_ref, wr_ref,
              hp_ref, hr_ref):
  f = accf_ref[0] + accf_ref[1]
  cnt = accc_ref[0] + accc_ref[1]
  mean = f / jnp.maximum(cnt, 1.0)
  h = jnp.maximum(mean + b1_ref[...] + xr_ref[...], 0.0)
  hp_ref[...] = jnp.dot(h, wl_ref[...], preferred_element_type=jnp.float32)
  hr_ref[...] = jnp.dot(h, wr_ref[...], preferred_element_type=jnp.float32)


def _tc3_body(accf_ref, accc_ref, hr_ref, b2_ref, onesbd_ref, out_ref):
  f = accf_ref[0] + accf_ref[1]
  cnt = accc_ref[0] + accc_ref[1]
  z = f / jnp.maximum(cnt, 1.0) + b2_ref[...] + hr_ref[...]
  off = jax.lax.broadcasted_iota(jnp.int32, z.shape, 1) % 16
  m = z
  for sft in (1, 2, 4, 8):
    right = pltpu.roll(m, 128 - sft, 1)
    left = pltpu.roll(m, sft, 1)
    m = jnp.where(off + sft < 16, jnp.maximum(m, right), m)
    m = jnp.where(off >= sft, jnp.maximum(m, left), m)
  zs = z - m
  e = jnp.exp(zs)
  s_all = jnp.dot(e, onesbd_ref[...], preferred_element_type=jnp.float32)
  out_ref[...] = zs - jnp.log(s_all)


def _block_diag8(w):
  return jnp.einsum("jJ,kf->jkJf", jnp.eye(8, dtype=w.dtype), w).reshape(
      8 * w.shape[0], 8 * w.shape[1])


def kernel(x, edge_index, W1l, b1, W1r, W2l, b2, W2r):
  n, f_in = x.shape
  e = edge_index.shape[1]
  h = W1l.shape[1]
  assert h == 16 and W2l.shape[1] == 16 and n % 8 == 0

  per_tile, rem = divmod(e, NW)
  assert rem == 0 and per_tile % 8 == 0

  npk = n // 8
  blk = npk
  grid = 1

  ei = edge_index.astype(jnp.int32).reshape(2 * e)
  x3 = x.reshape(npk, 8, f_in)
  Wd2l = _block_diag8(W2l)
  Wd2r = _block_diag8(W2r)
  b1t = jnp.tile(b1, 8).reshape(1, 8 * h)
  b2t = jnp.tile(b2, 8).reshape(1, 8 * h)

  pk_spec = pl.BlockSpec((blk, 128), lambda i: (i, 0))
  acc_spec = pl.BlockSpec((NC, blk, 128), lambda i: (0, i, 0))
  wd_spec = pl.BlockSpec((128, 128), lambda i: (0, 0))
  b_spec = pl.BlockSpec((1, 128), lambda i: (0, 0))
  x3_spec = pl.BlockSpec((blk, 8, f_in), lambda i: (i, 0, 0))
  w1_spec = pl.BlockSpec((f_in, h), lambda i: (0, 0))
  pk_shape = jax.ShapeDtypeStruct((npk, 128), jnp.float32)

  xp1, xr1 = pl.pallas_call(
      _tc1_body, grid=(grid,), in_specs=[x3_spec, w1_spec, w1_spec],
      out_specs=[pk_spec] * 2, out_shape=[pk_shape] * 2)(x3, W1l, W1r)

  agg1 = _make_sc_agg(n, per_tile, e, with_cnt=True)
  acc1, acc_cnt = agg1(xp1.reshape(n, h), ei)
  acc1 = acc1.reshape(NC, npk, 128)
  acc_cnt = acc_cnt.reshape(NC, npk, 128)

  hp2, hr2 = pl.pallas_call(
      _tc2_body, grid=(grid,),
      in_specs=[acc_spec, acc_spec, pk_spec, b_spec, wd_spec, wd_spec],
      out_specs=[pk_spec] * 2, out_shape=[pk_shape] * 2,
  )(acc1, acc_cnt, xr1, b1t, Wd2l, Wd2r)

  agg2 = _make_sc_agg(n, per_tile, e, with_cnt=False)
  (acc2,) = agg2(hp2.reshape(n, h), ei)
  acc2 = acc2.reshape(NC, npk, 128)

  onesbd = _block_diag8(jnp.ones((h, h), jnp.float32))
  out_pk = pl.pallas_call(
      _tc3_body, grid=(grid,),
      in_specs=[acc_spec, acc_spec, pk_spec, b_spec, wd_spec],
      out_specs=pk_spec, out_shape=pk_shape,
  )(acc2, acc_cnt, hr2, b2t, onesbd)

  return out_pk.reshape(n, h)
